# baseline (speedup 1.0000x reference)
import functools

import jax
import jax.numpy as jnp
from jax import lax
from jax.experimental import pallas as pl
from jax.experimental.pallas import tpu as pltpu
from jax.experimental.pallas import tpu_sc as plsc

D = 64
NC = 2
NS = 16
NW = NC * NS
C = 256
IDXW = 128
NG = C // IDXW


def _sc_gather(cat2, table, N):
    nchunk = N // (NW * C)
    mesh = plsc.VectorSubcoreMesh(core_axis_name="c", subcore_axis_name="s")

    @functools.partial(
        pl.kernel,
        out_type=jax.ShapeDtypeStruct((N, D), jnp.float32),
        mesh=mesh,
        compiler_params=pltpu.CompilerParams(use_tc_tiling_on_sc=False),
        scratch_types=(
            [pltpu.VMEM((NG, IDXW), jnp.int32) for _ in range(4)]
            + [pltpu.VMEM((C, D), jnp.float32) for _ in range(4)]
            + [pltpu.SemaphoreType.DMA for _ in range(2)]
            + [pltpu.SemaphoreType.DMA for _ in range(2)]
            + [pltpu.SemaphoreType.DMA for _ in range(4)]
        ),
    )
    def body(cat_hbm, table_hbm, out_hbm, *scratch):
        idxv = scratch[0:4]
        acc = scratch[4:8]
        sem_idx = scratch[8:10]
        sem_g = scratch[10:12]
        sem_out = scratch[12:16]

        cid = lax.axis_index("c")
        sid = lax.axis_index("s")
        wid = sid * NC + cid
        w0 = wid * nchunk

        def issue_idx(i, slot4, slot2):
            pltpu.async_copy(
                cat_hbm.at[pl.ds((w0 + i) * NG, NG)], idxv[slot4],
                sem_idx[slot2])

        def wait_idx(i, slot4, slot2):
            pltpu.make_async_copy(
                cat_hbm.at[pl.ds((w0 + i) * NG, NG)], idxv[slot4],
                sem_idx[slot2]).wait()

        def issue_gathers(slot4, slot2):
            for j in range(NG):
                pltpu.async_copy(
                    table_hbm.at[idxv[slot4].at[j]],
                    acc[slot4].at[pl.ds(j * IDXW, IDXW)],
                    sem_g[slot2])

        def wait_gathers(slot4, slot2):
            for j in range(NG):
                pltpu.make_async_copy(
                    table_hbm.at[idxv[slot4].at[j]],
                    acc[slot4].at[pl.ds(j * IDXW, IDXW)],
                    sem_g[slot2]).wait()

        def issue_out(i, slot4):
            pltpu.async_copy(
                acc[slot4], out_hbm.at[pl.ds((w0 + i) * C, C)],
                sem_out[slot4])

        def wait_out(i, slot4):
            pltpu.make_async_copy(
                acc[slot4], out_hbm.at[pl.ds((w0 + i) * C, C)],
                sem_out[slot4]).wait()

        issue_idx(0, 0, 0)
        issue_idx(1, 1, 1)
        wait_idx(0, 0, 0)
        issue_gathers(0, 0)

        def quad_body(i4, carry):
            for k in range(4):
                i = i4 * 4 + k
                s2, s2n = k % 2, (k + 1) % 2
                s4, s4n, s4nn = k, (k + 1) % 4, (k + 2) % 4

                @pl.when(i >= 3)
                def _():
                    wait_out(i - 3, s4n)

                @pl.when(i + 2 < nchunk)
                def _():
                    issue_idx(i + 2, s4nn, s2)

                @pl.when(i + 1 < nchunk)
                def _():
                    wait_idx(i + 1, s4n, s2n)
                    issue_gathers(s4n, s2n)

                wait_gathers(s4, s2)
                issue_out(i, s4)
            return carry

        lax.fori_loop(0, nchunk // 4, quad_body, 0)

        for t in (nchunk - 3, nchunk - 2, nchunk - 1):
            wait_out(t, t % 4)

    return body(cat2, table)


def _tc_masked_add(xv, emb3, mskv, S, B):
    SB = S // 8
    JB = B // 128
    nblk = SB * JB * 8

    def body(emb_ref, x_ref, m_ref, o_ref):
        e = emb_ref[0]
        keep = (m_ref[0, 0] == 0).astype(jnp.float32)
        em = e * keep[:, None]
        ident = (lax.broadcasted_iota(jnp.int32, (128, 128), 0)
                 == lax.broadcasted_iota(jnp.int32, (128, 128), 1)
                 ).astype(jnp.float32)
        et = lax.dot_general(
            em, ident, (((0,), (0,)), ((), ())),
            preferred_element_type=jnp.float32)
        x = x_ref[0, :, 0, :, :].reshape(D, 128)
        o_ref[0, :, 0, :, :] = (x + et).reshape(8, 8, 128)

    def emb_map(i):
        return (i, 0, 0)

    def x_map(i):
        sb = i // (JB * 8)
        j = (i // 8) % JB
        s8 = i % 8
        return (sb * 8 + s8, 0, j, 0, 0)

    grid = (nblk,)
    return pl.pallas_call(
        body,
        grid=grid,
        in_specs=[
            pl.BlockSpec((1, 128, D), emb_map),
            pl.BlockSpec((1, 8, 1, 8, 128), x_map),
            pl.BlockSpec((1, 1, 128), emb_map),
        ],
        out_specs=pl.BlockSpec((1, 8, 1, 8, 128), x_map),
        out_shape=jax.ShapeDtypeStruct((S, 8, JB, 8, 128), jnp.float32),
    )(emb3, xv, mskv)


def kernel(x, categories, mask_positions, table):
    B, S, d = x.shape
    N = B * S
    SB, JB = S // 8, B // 128

    xv = (x.transpose(1, 2, 0)
          .reshape(S, 8, 8, JB, 128)
          .transpose(0, 1, 3, 2, 4))
    cat2 = (categories.transpose(1, 0)
            .reshape(SB, 8, JB, 128)
            .transpose(0, 2, 1, 3)
            .reshape(N // 128, 128))
    mskv = (mask_positions.reshape(B, S)
            .transpose(1, 0)
            .reshape(SB, 8, JB, 128)
            .transpose(0, 2, 1, 3)
            .reshape(N // 128, 1, 128))

    emb = _sc_gather(cat2, table, N)
    emb3 = emb.reshape(N // 128, 128, d)

    out5 = _tc_masked_add(xv, emb3, mskv, S, B)
    out = (out5.transpose(0, 1, 3, 2, 4)
           .reshape(S, d, B)
           .transpose(2, 0, 1))
    return out


# speedup vs baseline: 2.8000x; 2.8000x over previous
import functools

import jax
import jax.numpy as jnp
from jax import lax
from jax.experimental import pallas as pl
from jax.experimental.pallas import tpu as pltpu
from jax.experimental.pallas import tpu_sc as plsc

D = 64
NC = 2
NS = 16
NW = NC * NS
C = 256
IDXW = 128
NG = C // IDXW


def _sc_gather(cat2, table, N):
    nchunk = N // (NW * C)
    mesh = plsc.VectorSubcoreMesh(core_axis_name="c", subcore_axis_name="s")

    @functools.partial(
        pl.kernel,
        out_type=jax.ShapeDtypeStruct((N, D), jnp.float32),
        mesh=mesh,
        compiler_params=pltpu.CompilerParams(use_tc_tiling_on_sc=False),
        scratch_types=(
            [pltpu.VMEM((NG, IDXW), jnp.int32) for _ in range(4)]
            + [pltpu.VMEM((C, D), jnp.float32) for _ in range(4)]
            + [pltpu.SemaphoreType.DMA for _ in range(2)]
            + [pltpu.SemaphoreType.DMA for _ in range(2)]
            + [pltpu.SemaphoreType.DMA for _ in range(4)]
        ),
    )
    def body(cat_hbm, table_hbm, out_hbm, *scratch):
        idxv = scratch[0:4]
        acc = scratch[4:8]
        sem_idx = scratch[8:10]
        sem_g = scratch[10:12]
        sem_out = scratch[12:16]

        cid = lax.axis_index("c")
        sid = lax.axis_index("s")
        wid = sid * NC + cid
        w0 = wid * nchunk

        def issue_idx(i, slot4, slot2):
            pltpu.async_copy(
                cat_hbm.at[pl.ds((w0 + i) * NG, NG)], idxv[slot4],
                sem_idx[slot2])

        def wait_idx(i, slot4, slot2):
            pltpu.make_async_copy(
                cat_hbm.at[pl.ds((w0 + i) * NG, NG)], idxv[slot4],
                sem_idx[slot2]).wait()

        def issue_gathers(slot4, slot2):
            for j in range(NG):
                pltpu.async_copy(
                    table_hbm.at[idxv[slot4].at[j]],
                    acc[slot4].at[pl.ds(j * IDXW, IDXW)],
                    sem_g[slot2])

        def wait_gathers(slot4, slot2):
            for j in range(NG):
                pltpu.make_async_copy(
                    table_hbm.at[idxv[slot4].at[j]],
                    acc[slot4].at[pl.ds(j * IDXW, IDXW)],
                    sem_g[slot2]).wait()

        def issue_out(i, slot4):
            pltpu.async_copy(
                acc[slot4], out_hbm.at[pl.ds((w0 + i) * C, C)],
                sem_out[slot4])

        def wait_out(i, slot4):
            pltpu.make_async_copy(
                acc[slot4], out_hbm.at[pl.ds((w0 + i) * C, C)],
                sem_out[slot4]).wait()

        issue_idx(0, 0, 0)
        issue_idx(1, 1, 1)
        wait_idx(0, 0, 0)
        issue_gathers(0, 0)

        def quad_body(i4, carry):
            for k in range(4):
                i = i4 * 4 + k
                s2, s2n = k % 2, (k + 1) % 2
                s4, s4n, s4nn = k, (k + 1) % 4, (k + 2) % 4

                @pl.when(i >= 3)
                def _():
                    wait_out(i - 3, s4n)

                @pl.when(i + 2 < nchunk)
                def _():
                    issue_idx(i + 2, s4nn, s2)

                @pl.when(i + 1 < nchunk)
                def _():
                    wait_idx(i + 1, s4n, s2n)
                    issue_gathers(s4n, s2n)

                wait_gathers(s4, s2)
                issue_out(i, s4)
            return carry

        lax.fori_loop(0, nchunk // 4, quad_body, 0)

        for t in (nchunk - 3, nchunk - 2, nchunk - 1):
            wait_out(t, t % 4)

    return body(cat2, table)


def _tc_masked_add(xv, emb3, mskv, S, B):
    SB = S // 8
    JB = B // 128
    nblk = SB * JB * 8

    def body(emb_ref, x_ref, m_ref, o_ref):
        ident = (lax.broadcasted_iota(jnp.int32, (128, 128), 0)
                 == lax.broadcasted_iota(jnp.int32, (128, 128), 1)
                 ).astype(jnp.float32)
        for s8 in range(8):
            e = emb_ref[0, pl.ds(s8 * 128, 128), :]
            keep = (m_ref[0, s8] == 0).astype(jnp.float32)
            em = e * keep[:, None]
            et = lax.dot_general(
                em, ident, (((0,), (0,)), ((), ())),
                preferred_element_type=jnp.float32)
            x = x_ref[s8, :, 0, :, :].reshape(D, 128)
            o_ref[s8, :, 0, :, :] = (x + et).reshape(8, 8, 128)

    def emb_map(i):
        return (i, 0, 0)

    def x_map(i):
        return (i // JB, 0, i % JB, 0, 0)

    grid = (nblk // 8,)
    return pl.pallas_call(
        body,
        grid=grid,
        in_specs=[
            pl.BlockSpec((1, 1024, D), emb_map),
            pl.BlockSpec((8, 8, 1, 8, 128), x_map),
            pl.BlockSpec((1, 8, 128), emb_map),
        ],
        out_specs=pl.BlockSpec((8, 8, 1, 8, 128), x_map),
        out_shape=jax.ShapeDtypeStruct((S, 8, JB, 8, 128), jnp.float32),
    )(emb3, xv, mskv)


def kernel(x, categories, mask_positions, table):
    B, S, d = x.shape
    N = B * S
    SB, JB = S // 8, B // 128

    xv = (x.transpose(1, 2, 0)
          .reshape(S, 8, 8, JB, 128)
          .transpose(0, 1, 3, 2, 4))
    cat2 = (categories.transpose(1, 0)
            .reshape(SB, 8, JB, 128)
            .transpose(0, 2, 1, 3)
            .reshape(N // 128, 128))
    mskv = (mask_positions.reshape(B, S)
            .transpose(1, 0)
            .reshape(SB, 8, JB, 128)
            .transpose(0, 2, 1, 3)
            .reshape(N // 1024, 8, 128))

    emb = _sc_gather(cat2, table, N)
    emb3 = emb.reshape(N // 1024, 1024, d)

    out5 = _tc_masked_add(xv, emb3, mskv, S, B)
    out = (out5.transpose(0, 1, 3, 2, 4)
           .reshape(S, d, B)
           .transpose(2, 0, 1))
    return out
